# baseline (speedup 1.0000x reference)
import functools

import jax
import jax.numpy as jnp
from jax import lax
from jax.experimental import pallas as pl
from jax.experimental.pallas import tpu as pltpu
from jax.experimental.pallas import tpu_sc as plsc

N = 10000
B = 10
NPG = 1000
E = 320000
H = 128
SK = 64
C = 10
K1 = 500
K2 = 250

NCORE = 2
NSUB = 16
NWORK = NCORE * NSUB
CH = E // NWORK
M = 80
NB = CH // M
HALF = N // NCORE
SLAB = 5120
RPS = SLAB // NSUB
NDEG = 10240
DPS = NDEG // NSUB
NB2 = 128
MP = 64
NBP = NB2 * M // MP

_MESH = plsc.VectorSubcoreMesh(core_axis_name="c", subcore_axis_name="s")


def _edge_pass_body(act_hbm, src_hbm, dst_hbm, deg_hbm, srcc_hbm, dstlc_hbm,
                    cnt_hbm, act_v, src_v, dst_v, srcc_f, dstlc_f, diag_a,
                    diag_b, zb_v, cnt_v, deg_sh, dsem_a, dsem_b):
    c = lax.axis_index("c")
    s = lax.axis_index("s")

    def zb_zero(i, _):
        zb_v[i, pl.ds(0, 16)] = jnp.zeros((16,), jnp.float32)
        return 0
    lax.fori_loop(0, DPS, zb_zero, 0)
    pltpu.sync_copy(zb_v, deg_sh.at[pl.ds(s * DPS, DPS)])

    def dg_zero(i, _):
        diag_a[i, pl.ds(0, 16)] = jnp.zeros((16,), jnp.float32)
        diag_b[i, pl.ds(0, 16)] = jnp.zeros((16,), jnp.float32)
        return 0
    lax.fori_loop(0, M, dg_zero, 0)
    plsc.subcore_barrier()

    pltpu.sync_copy(src_hbm.at[c, s], src_v)
    pltpu.sync_copy(dst_hbm.at[c, s], dst_v)
    pltpu.sync_copy(act_hbm, act_v)

    lane = lax.iota(jnp.int32, 16)

    def do_row(r, cur, dv):
        for j in range(M // 16):
            s16 = src_v[r, pl.ds(j * 16, 16)]
            d16 = dst_v[r, pl.ds(j * 16, 16)]
            a_s = plsc.load_gather(act_v, [s16])
            a_d = plsc.load_gather(act_v, [d16])
            w = a_s * a_d
            plsc.store_scatter(dv, [j * 16 + lane, lane], w)
            msk = w > 0.0
            mi = jnp.where(msk, 1, 0).astype(jnp.int32)
            ps = plsc.cumsum(mi)
            idx = cur + ps - 1
            dl = jnp.where(d16 >= HALF, d16 - HALF, d16)
            plsc.store_scatter(srcc_f, [idx], s16, mask=msk)
            plsc.store_scatter(dstlc_f, [idx], dl, mask=msk)
            cur = cur + jnp.sum(mi)
        return cur

    cur0 = do_row(0, jnp.int32(0), diag_a)
    pltpu.async_copy(diag_a, deg_sh.at[src_v.at[0]], dsem_a, add=True)

    def pair_body(p, cur):
        r1 = 2 * p + 1

        @pl.when(p > 0)
        def _():
            pltpu.make_async_copy(diag_b, deg_sh.at[src_v.at[r1]],
                                  dsem_b).wait()
        cur = do_row(r1, cur, diag_b)
        pltpu.async_copy(diag_b, deg_sh.at[src_v.at[r1]], dsem_b, add=True)
        r2 = 2 * p + 2
        pltpu.make_async_copy(diag_a, deg_sh.at[src_v.at[r2]], dsem_a).wait()
        cur = do_row(r2, cur, diag_a)
        pltpu.async_copy(diag_a, deg_sh.at[src_v.at[r2]], dsem_a, add=True)
        return cur

    cnt = lax.fori_loop(0, (NB - 1) // 2, pair_body, cur0)
    pltpu.make_async_copy(diag_a, deg_sh.at[src_v.at[0]], dsem_a).wait()
    pltpu.make_async_copy(diag_b, deg_sh.at[src_v.at[0]], dsem_b).wait()
    dumdl = HALF + (lane & 7)
    zsrc = jnp.zeros((16,), jnp.int32)
    for j in range(MP // 16):
        plsc.store_scatter(srcc_f, [cnt + j * 16 + lane], zsrc)
        plsc.store_scatter(dstlc_f, [cnt + j * 16 + lane], dumdl)
    cnt_v[pl.ds(0, 16)] = jnp.full((16,), cnt, jnp.int32)
    pltpu.sync_copy(cnt_v, cnt_hbm.at[c, s])
    pltpu.sync_copy(srcc_f, srcc_hbm.at[c, s])
    pltpu.sync_copy(dstlc_f, dstlc_hbm.at[c, s])
    plsc.subcore_barrier()
    pltpu.sync_copy(deg_sh.at[pl.ds(s * DPS, DPS)],
                    deg_hbm.at[c, pl.ds(s * DPS, DPS)])


_edge_pass = pl.kernel(
    _edge_pass_body,
    out_type=(
        jax.ShapeDtypeStruct((NCORE, NDEG, 16), jnp.float32),
        jax.ShapeDtypeStruct((NCORE, NSUB, NB2 * M), jnp.int32),
        jax.ShapeDtypeStruct((NCORE, NSUB, NB2 * M), jnp.int32),
        jax.ShapeDtypeStruct((NCORE, NSUB, 16), jnp.int32),
    ),
    mesh=_MESH,
    compiler_params=pltpu.CompilerParams(
        needs_layout_passes=False, use_tc_tiling_on_sc=False),
    scratch_types=[
        pltpu.VMEM((N,), jnp.float32),
        pltpu.VMEM((NB, M), jnp.int32),
        pltpu.VMEM((NB, M), jnp.int32),
        pltpu.VMEM((NB2 * M,), jnp.int32),
        pltpu.VMEM((NB2 * M,), jnp.int32),
        pltpu.VMEM((M, 16), jnp.float32),
        pltpu.VMEM((M, 16), jnp.float32),
        pltpu.VMEM((DPS, 16), jnp.float32),
        pltpu.VMEM((16,), jnp.int32),
        pltpu.VMEM_SHARED((NDEG, 16), jnp.float32),
        pltpu.SemaphoreType.DMA,
        pltpu.SemaphoreType.DMA,
    ],
)


def _propagate_body(hs_hbm, srcc_hbm, dstlc_hbm, cnt_hbm, acc_hbm,
                    src_v, dlx_v, cnt_v, b0, b1, b2, b3, b4, b5, b6, b7,
                    acc_sh, s0, s1, s2, s3, s4, s5, s6, s7):
    c = lax.axis_index("c")
    s = lax.axis_index("s")
    bufs = (b0, b1, b2, b3, b4, b5, b6, b7)
    sems = (s0, s1, s2, s3, s4, s5, s6, s7)

    def z_body(i, _):
        for k in range(8):
            b0[i, pl.ds(k * 16, 16)] = jnp.zeros((16,), jnp.float32)
        return 0
    lax.fori_loop(0, MP, z_body, 0)
    base = s * RPS
    for q in range(RPS // MP):
        pltpu.sync_copy(b0, acc_sh.at[pl.ds(base + q * MP, MP)])
    plsc.subcore_barrier()

    pltpu.sync_copy(srcc_hbm.at[c, s], src_v)
    pltpu.sync_copy(dstlc_hbm.at[c, s], dlx_v)
    pltpu.sync_copy(cnt_hbm.at[c, s], cnt_v)
    cnt = jnp.max(cnt_v[pl.ds(0, 16)])

    for j in range(8):
        @pl.when(j * MP < cnt)
        def _(j=j):
            pltpu.async_copy(hs_hbm.at[src_v.at[j]], bufs[j], sems[j])

    def group(g, _):
        for j in range(8):
            b = g * 8 + j

            @pl.when(b * MP < cnt)
            def _(b=b, j=j):
                pltpu.make_async_copy(hs_hbm.at[src_v.at[b]], bufs[j],
                                      sems[j]).wait()
                pltpu.sync_copy(bufs[j], acc_sh.at[dlx_v.at[b]], add=True)
                nxt = b + 8

                @pl.when(nxt * MP < cnt)
                def _():
                    pltpu.async_copy(hs_hbm.at[src_v.at[nxt]], bufs[j],
                                     sems[j])
        return 0

    lax.fori_loop(0, NBP // 8, group, 0)
    plsc.subcore_barrier()
    pltpu.sync_copy(acc_sh.at[pl.ds(base, RPS)],
                    acc_hbm.at[c, pl.ds(base, RPS)])


_propagate = pl.kernel(
    _propagate_body,
    out_type=jax.ShapeDtypeStruct((NCORE, SLAB, H), jnp.float32),
    mesh=_MESH,
    compiler_params=pltpu.CompilerParams(
        needs_layout_passes=False, use_tc_tiling_on_sc=False),
    scratch_types=[
        pltpu.VMEM((NBP, MP), jnp.int32),
        pltpu.VMEM((NBP, MP), jnp.int32),
        pltpu.VMEM((16,), jnp.int32),
        pltpu.VMEM((MP, H), jnp.float32),
        pltpu.VMEM((MP, H), jnp.float32),
        pltpu.VMEM((MP, H), jnp.float32),
        pltpu.VMEM((MP, H), jnp.float32),
        pltpu.VMEM((MP, H), jnp.float32),
        pltpu.VMEM((MP, H), jnp.float32),
        pltpu.VMEM((MP, H), jnp.float32),
        pltpu.VMEM((MP, H), jnp.float32),
        pltpu.VMEM_SHARED((SLAB, H), jnp.float32),
        pltpu.SemaphoreType.DMA,
        pltpu.SemaphoreType.DMA,
        pltpu.SemaphoreType.DMA,
        pltpu.SemaphoreType.DMA,
        pltpu.SemaphoreType.DMA,
        pltpu.SemaphoreType.DMA,
        pltpu.SemaphoreType.DMA,
        pltpu.SemaphoreType.DMA,
    ],
)


def _mm_body(x_ref, w_ref, xw_ref):
    xw_ref[...] = jnp.dot(x_ref[...], w_ref[...],
                          preferred_element_type=jnp.float32)


def _mm(x, w):
    r = x.shape[0] // NPG
    return pl.pallas_call(
        _mm_body,
        grid=(r,),
        in_specs=[
            pl.BlockSpec((NPG, x.shape[1]), lambda g: (g, 0)),
            pl.BlockSpec(w.shape, lambda g: (0, 0)),
        ],
        out_specs=pl.BlockSpec((NPG, H), lambda g: (g, 0)),
        out_shape=jax.ShapeDtypeStruct((x.shape[0], H), jnp.float32),
    )(x, w)


def _scale_body(xw_ref, degp_ref, xws_ref, deg_ref):
    dp = jnp.sum(degp_ref[...], axis=0)
    deg0 = jnp.sum(dp, axis=1, keepdims=True)
    di = lax.rsqrt(deg0 + 1.0)
    xws_ref[...] = di * xw_ref[...]
    deg_ref[...] = deg0


def _scale(xw, degp):
    r = xw.shape[0] // NPG
    return pl.pallas_call(
        _scale_body,
        grid=(r,),
        in_specs=[
            pl.BlockSpec((NPG, H), lambda g: (g, 0)),
            pl.BlockSpec((NCORE, NPG, 16), lambda g: (0, g, 0)),
        ],
        out_specs=[
            pl.BlockSpec((NPG, H), lambda g: (g, 0)),
            pl.BlockSpec((NPG, 1), lambda g: (g, 0)),
        ],
        out_shape=[
            jax.ShapeDtypeStruct((xw.shape[0], H), jnp.float32),
            jax.ShapeDtypeStruct((xw.shape[0], 1), jnp.float32),
        ],
    )(xw, degp)


def _gcn_epi_body(acc_ref, xws_ref, deg_ref, b_ref, h_ref, hs_ref):
    deg0 = deg_ref[...]
    di1 = lax.rsqrt(deg0 + 1.0)
    di0 = jnp.where(deg0 > 0.0, lax.rsqrt(jnp.maximum(deg0, 1e-30)), 0.0)
    h = jnp.maximum(di1 * (acc_ref[0] + xws_ref[...]) + b_ref[...], 0.0)
    h_ref[...] = h
    hs_ref[...] = di0 * h


def _gcn_epilogue(acc, xws, deg0, b):
    return pl.pallas_call(
        _gcn_epi_body,
        grid=(B,),
        in_specs=[
            pl.BlockSpec((1, NPG, H), lambda g: (g // 5, g % 5, 0)),
            pl.BlockSpec((NPG, H), lambda g: (g, 0)),
            pl.BlockSpec((NPG, 1), lambda g: (g, 0)),
            pl.BlockSpec((1, H), lambda g: (0, 0)),
        ],
        out_specs=[
            pl.BlockSpec((NPG, H), lambda g: (g, 0)),
            pl.BlockSpec((NPG, H), lambda g: (g, 0)),
        ],
        out_shape=[
            jax.ShapeDtypeStruct((N, H), jnp.float32),
            jax.ShapeDtypeStruct((N, H), jnp.float32),
        ],
    )(acc, xws, deg0, b)


def _score_pool_body(k, h_ref, acc_ref, deg_ref, mcol_ref, mrow_ref,
                     arow_ref, acol_ref, gmp_ref, gap_ref):
    deg0 = deg_ref[...]
    di0 = jnp.where(deg0 > 0.0, lax.rsqrt(jnp.maximum(deg0, 1e-30)), 0.0)
    h = h_ref[...]
    prop = jnp.abs(h - di0 * acc_ref[0])
    ones_row = jnp.ones((1, H), jnp.float32)
    s_col = jnp.sum(prop, axis=1, keepdims=True)
    s_row = lax.dot_general(ones_row, prop, (((1,), (1,)), ((), ())),
                            preferred_element_type=jnp.float32)
    ninf = jnp.float32(-jnp.inf)
    s_col = jnp.where(mcol_ref[...] > 0.0, s_col, ninf)
    s_row = jnp.where(mrow_ref[0] > 0.0, s_row, ninf)
    i0 = lax.broadcasted_iota(jnp.int32, (NPG, NPG), 0)
    i1 = lax.broadcasted_iota(jnp.int32, (NPG, NPG), 1)
    beats = (s_row > s_col) | ((s_row == s_col) & (i1 < i0))
    pf = jnp.where(beats, 1.0, 0.0)
    ones_col = jnp.ones((NPG, 1), jnp.float32)
    rank_col = jnp.dot(pf, ones_col, preferred_element_type=jnp.float32)
    a_col = jnp.where(rank_col < float(k), 1.0, 0.0)
    ones_1n = jnp.ones((1, NPG), jnp.float32)
    colsum = lax.dot_general(ones_1n, pf, (((1,), (0,)), ((), ())),
                             preferred_element_type=jnp.float32)
    rank_row = float(NPG - 1) - colsum
    a_row = jnp.where(rank_row < float(k), 1.0, 0.0)
    arow_ref[0] = a_row
    acol_ref[...] = a_col
    gmp_ref[0] = jnp.max(jnp.where(a_col > 0.0, h, ninf), axis=0,
                         keepdims=True)
    gap_ref[0] = jnp.sum(jnp.where(a_col > 0.0, h, 0.0), axis=0,
                         keepdims=True) * (1.0 / k)


def _score_pool(k, h, acc, deg0, mcol, mrow):
    return pl.pallas_call(
        functools.partial(_score_pool_body, k),
        grid=(B,),
        in_specs=[
            pl.BlockSpec((NPG, H), lambda g: (g, 0)),
            pl.BlockSpec((1, NPG, H), lambda g: (g // 5, g % 5, 0)),
            pl.BlockSpec((NPG, 1), lambda g: (g, 0)),
            pl.BlockSpec((NPG, 1), lambda g: (g, 0)),
            pl.BlockSpec((1, 1, NPG), lambda g: (g, 0, 0)),
        ],
        out_specs=[
            pl.BlockSpec((1, 1, NPG), lambda g: (g, 0, 0)),
            pl.BlockSpec((NPG, 1), lambda g: (g, 0)),
            pl.BlockSpec((1, 1, H), lambda g: (g, 0, 0)),
            pl.BlockSpec((1, 1, H), lambda g: (g, 0, 0)),
        ],
        out_shape=[
            jax.ShapeDtypeStruct((B, 1, NPG), jnp.float32),
            jax.ShapeDtypeStruct((N, 1), jnp.float32),
            jax.ShapeDtypeStruct((B, 1, H), jnp.float32),
            jax.ShapeDtypeStruct((B, 1, H), jnp.float32),
        ],
    )(h, acc, deg0, mcol, mrow)


def _final_gcn_body(acc_ref, xws_ref, deg_ref, b_ref, acol_ref,
                    gmp_ref, gap_ref):
    di1 = lax.rsqrt(deg_ref[...] + 1.0)
    h = jnp.maximum(di1 * (acc_ref[0] + xws_ref[...]) + b_ref[...], 0.0)
    a_col = acol_ref[...]
    ninf = jnp.float32(-jnp.inf)
    gmp_ref[0] = jnp.max(jnp.where(a_col > 0.0, h, ninf), axis=0,
                         keepdims=True)
    gap_ref[0] = jnp.sum(jnp.where(a_col > 0.0, h, 0.0), axis=0,
                         keepdims=True) * (1.0 / K2)


def _final_gcn(acc, xws, deg0, b, acol):
    return pl.pallas_call(
        _final_gcn_body,
        grid=(B,),
        in_specs=[
            pl.BlockSpec((1, NPG, H), lambda g: (g // 5, g % 5, 0)),
            pl.BlockSpec((NPG, H), lambda g: (g, 0)),
            pl.BlockSpec((NPG, 1), lambda g: (g, 0)),
            pl.BlockSpec((1, H), lambda g: (0, 0)),
            pl.BlockSpec((NPG, 1), lambda g: (g, 0)),
        ],
        out_specs=[
            pl.BlockSpec((1, 1, H), lambda g: (g, 0, 0)),
            pl.BlockSpec((1, 1, H), lambda g: (g, 0, 0)),
        ],
        out_shape=[
            jax.ShapeDtypeStruct((B, 1, H), jnp.float32),
            jax.ShapeDtypeStruct((B, 1, H), jnp.float32),
        ],
    )(acc, xws, deg0, b, acol)


def _head_body(g1, a1, g2, a2, g3, a3, skew_ref, wsk_ref, bsk_ref,
               wl1_ref, bl1_ref, wl2_ref, bl2_ref, wl3_ref, bl3_ref, out_ref):
    r = jnp.maximum
    zmx = r(g1[...], 0.0) + r(g2[...], 0.0) + r(g3[...], 0.0)
    zmn = r(a1[...], 0.0) + r(a2[...], 0.0) + r(a3[...], 0.0)
    xskew = r(jnp.dot(skew_ref[...], wsk_ref[...],
                      preferred_element_type=jnp.float32) + bsk_ref[...], 0.0)
    z = jnp.concatenate([zmx, zmn, xskew], axis=1)
    z = r(jnp.dot(z, wl1_ref[...], preferred_element_type=jnp.float32)
          + bl1_ref[...], 0.0)
    z = r(jnp.dot(z, wl2_ref[...], preferred_element_type=jnp.float32)
          + bl2_ref[...], 0.0)
    o = jnp.dot(z, wl3_ref[...], preferred_element_type=jnp.float32) \
        + bl3_ref[...]
    m = jnp.max(o, axis=1, keepdims=True)
    lse = jnp.log(jnp.sum(jnp.exp(o - m), axis=1, keepdims=True)) + m
    out_ref[...] = o - lse


def _head(g1, a1, g2, a2, g3, a3, skew, wsk, bsk, wl1, bl1, wl2, bl2,
          wl3, bl3):
    args = (g1, a1, g2, a2, g3, a3, skew, wsk, bsk, wl1, bl1, wl2, bl2,
            wl3, bl3)
    return pl.pallas_call(
        _head_body,
        in_specs=[pl.BlockSpec(a.shape, lambda: (0, 0)) for a in args],
        out_specs=pl.BlockSpec((B, C), lambda: (0, 0)),
        out_shape=jax.ShapeDtypeStruct((B, C), jnp.float32),
    )(*args)


def kernel(x, edge_index, batch, skew, W1, b1, Wsk, bsk, W2, b2, W3, b3,
           Wl1, bl1, Wl2, bl2, Wl3, bl3):
    del batch
    src = edge_index[0]
    dst = edge_index[1]
    src_r = src.reshape(NCORE, NSUB, NB, M)
    dst_r = dst.reshape(NCORE, NSUB, NB, M)
    ones_n = jnp.ones((N,), jnp.float32)
    ones_col = jnp.ones((N, 1), jnp.float32)
    ones_row = jnp.ones((B, 1, NPG), jnp.float32)
    b1r = b1.reshape(1, H)
    b2r = b2.reshape(1, H)
    b3r = b3.reshape(1, H)
    r4 = lambda t: t.reshape(NCORE, NSUB, NBP, MP)

    xw1 = _mm(x, W1)
    degp1, srcc1, dstlc1, cnt1 = _edge_pass(ones_n, src_r, dst_r)
    srcc1, dstlc1 = r4(srcc1), r4(dstlc1)
    xw1s, deg01 = _scale(xw1, degp1)
    acc1 = _propagate(xw1s, srcc1, dstlc1, cnt1)
    h1, h1s = _gcn_epilogue(acc1, xw1s, deg01, b1r)
    acc2 = _propagate(h1s, srcc1, dstlc1, cnt1)
    a2row, a2col, gmp1, gap1 = _score_pool(K1, h1, acc2, deg01,
                                           ones_col, ones_row)

    xw2 = _mm(h1, W2)
    degp2, srcc2, dstlc2, cnt2 = _edge_pass(a2col.reshape(N), src_r, dst_r)
    srcc2, dstlc2 = r4(srcc2), r4(dstlc2)
    xw2s, deg02 = _scale(xw2, degp2)
    acc3 = _propagate(xw2s, srcc2, dstlc2, cnt2)
    h2, h2s = _gcn_epilogue(acc3, xw2s, deg02, b2r)
    acc4 = _propagate(h2s, srcc2, dstlc2, cnt2)
    a3row, a3col, gmp2, gap2 = _score_pool(K2, h2, acc4, deg02, a2col, a2row)

    xw3 = _mm(h2, W3)
    degp3, srcc3, dstlc3, cnt3 = _edge_pass(a3col.reshape(N), src_r, dst_r)
    srcc3, dstlc3 = r4(srcc3), r4(dstlc3)
    xw3s, deg03 = _scale(xw3, degp3)
    acc5 = _propagate(xw3s, srcc3, dstlc3, cnt3)
    gmp3, gap3 = _final_gcn(acc5, xw3s, deg03, b3r, a3col)

    rs = lambda t: t.reshape(B, H)
    return _head(rs(gmp1), rs(gap1), rs(gmp2), rs(gap2), rs(gmp3), rs(gap3), skew,
                 Wsk, bsk.reshape(1, H), Wl1, bl1.reshape(1, H),
                 Wl2, bl2.reshape(1, SK), Wl3, bl3.reshape(1, C))


# speedup vs baseline: 1.1330x; 1.1330x over previous
import functools

import jax
import jax.numpy as jnp
from jax import lax
from jax.experimental import pallas as pl
from jax.experimental.pallas import tpu as pltpu
from jax.experimental.pallas import tpu_sc as plsc

N = 10000
B = 10
NPG = 1000
E = 320000
H = 128
SK = 64
C = 10
K1 = 500
K2 = 250

NCORE = 2
NSUB = 16
NWORK = NCORE * NSUB
CH = E // NWORK
M = 80
NB = CH // M
HALF = N // NCORE
SLAB = 5120
RPS = SLAB // NSUB
NDEG = 10240
DPS = NDEG // NSUB
NB2 = 128
MP = 80
NBP = NB2 * M // MP

_MESH = plsc.VectorSubcoreMesh(core_axis_name="c", subcore_axis_name="s")


def _edge_pass_body(act_hbm, src_hbm, dst_hbm, deg_hbm, srcc_hbm, dstlc_hbm,
                    cnt_hbm, act_v, src_v, dst_v, srcc_f, dstlc_f, diag_a,
                    diag_b, zb_v, cnt_v, deg_sh, dsem_a, dsem_b):
    c = lax.axis_index("c")
    s = lax.axis_index("s")

    def zb_zero(i, _):
        zb_v[i, pl.ds(0, 16)] = jnp.zeros((16,), jnp.float32)
        return 0
    lax.fori_loop(0, DPS, zb_zero, 0)
    pltpu.sync_copy(zb_v, deg_sh.at[pl.ds(s * DPS, DPS)])

    def dg_zero(i, _):
        diag_a[i, pl.ds(0, 16)] = jnp.zeros((16,), jnp.float32)
        diag_b[i, pl.ds(0, 16)] = jnp.zeros((16,), jnp.float32)
        return 0
    lax.fori_loop(0, M, dg_zero, 0)
    plsc.subcore_barrier()

    pltpu.sync_copy(src_hbm.at[c, s], src_v)
    pltpu.sync_copy(dst_hbm.at[c, s], dst_v)
    pltpu.sync_copy(act_hbm, act_v)

    lane = lax.iota(jnp.int32, 16)

    def do_row(r, cur, dv):
        for j in range(M // 16):
            s16 = src_v[r, pl.ds(j * 16, 16)]
            d16 = dst_v[r, pl.ds(j * 16, 16)]
            a_s = plsc.load_gather(act_v, [s16])
            a_d = plsc.load_gather(act_v, [d16])
            w = a_s * a_d
            plsc.store_scatter(dv, [j * 16 + lane, lane], w)
            msk = w > 0.0
            mi = jnp.where(msk, 1, 0).astype(jnp.int32)
            ps = plsc.cumsum(mi)
            idx = cur + ps - 1
            dl = jnp.where(d16 >= HALF, d16 - HALF, d16)
            plsc.store_scatter(srcc_f, [idx], s16, mask=msk)
            plsc.store_scatter(dstlc_f, [idx], dl, mask=msk)
            cur = cur + jnp.sum(mi)
        return cur

    cur0 = do_row(0, jnp.int32(0), diag_a)
    pltpu.async_copy(diag_a, deg_sh.at[src_v.at[0]], dsem_a, add=True)

    def pair_body(p, cur):
        r1 = 2 * p + 1

        @pl.when(p > 0)
        def _():
            pltpu.make_async_copy(diag_b, deg_sh.at[src_v.at[r1]],
                                  dsem_b).wait()
        cur = do_row(r1, cur, diag_b)
        pltpu.async_copy(diag_b, deg_sh.at[src_v.at[r1]], dsem_b, add=True)
        r2 = 2 * p + 2
        pltpu.make_async_copy(diag_a, deg_sh.at[src_v.at[r2]], dsem_a).wait()
        cur = do_row(r2, cur, diag_a)
        pltpu.async_copy(diag_a, deg_sh.at[src_v.at[r2]], dsem_a, add=True)
        return cur

    cnt = lax.fori_loop(0, (NB - 1) // 2, pair_body, cur0)
    pltpu.make_async_copy(diag_a, deg_sh.at[src_v.at[0]], dsem_a).wait()
    pltpu.make_async_copy(diag_b, deg_sh.at[src_v.at[0]], dsem_b).wait()
    dumdl = HALF + (lane & 7)
    zsrc = jnp.zeros((16,), jnp.int32)
    for j in range(MP // 16):
        plsc.store_scatter(srcc_f, [cnt + j * 16 + lane], zsrc)
        plsc.store_scatter(dstlc_f, [cnt + j * 16 + lane], dumdl)
    cnt_v[pl.ds(0, 16)] = jnp.full((16,), cnt, jnp.int32)
    pltpu.sync_copy(cnt_v, cnt_hbm.at[c, s])
    pltpu.sync_copy(srcc_f, srcc_hbm.at[c, s])
    pltpu.sync_copy(dstlc_f, dstlc_hbm.at[c, s])
    plsc.subcore_barrier()
    pltpu.sync_copy(deg_sh.at[pl.ds(s * DPS, DPS)],
                    deg_hbm.at[c, pl.ds(s * DPS, DPS)])


_edge_pass = pl.kernel(
    _edge_pass_body,
    out_type=(
        jax.ShapeDtypeStruct((NCORE, NDEG, 16), jnp.float32),
        jax.ShapeDtypeStruct((NCORE, NSUB, NB2 * M), jnp.int32),
        jax.ShapeDtypeStruct((NCORE, NSUB, NB2 * M), jnp.int32),
        jax.ShapeDtypeStruct((NCORE, NSUB, 16), jnp.int32),
    ),
    mesh=_MESH,
    compiler_params=pltpu.CompilerParams(
        needs_layout_passes=False, use_tc_tiling_on_sc=False),
    scratch_types=[
        pltpu.VMEM((N,), jnp.float32),
        pltpu.VMEM((NB, M), jnp.int32),
        pltpu.VMEM((NB, M), jnp.int32),
        pltpu.VMEM((NB2 * M,), jnp.int32),
        pltpu.VMEM((NB2 * M,), jnp.int32),
        pltpu.VMEM((M, 16), jnp.float32),
        pltpu.VMEM((M, 16), jnp.float32),
        pltpu.VMEM((DPS, 16), jnp.float32),
        pltpu.VMEM((16,), jnp.int32),
        pltpu.VMEM_SHARED((NDEG, 16), jnp.float32),
        pltpu.SemaphoreType.DMA,
        pltpu.SemaphoreType.DMA,
    ],
)


def _propagate_body(hs_hbm, srcc_hbm, dstlc_hbm, cnt_hbm, acc_hbm,
                    src_v, dlx_v, cnt_v, b0, b1, b2, b3, b4, b5,
                    acc_sh, s0, s1, s2, s3, s4, s5):
    c = lax.axis_index("c")
    s = lax.axis_index("s")
    bufs = (b0, b1, b2, b3, b4, b5)
    sems = (s0, s1, s2, s3, s4, s5)

    def z_body(i, _):
        for k in range(8):
            b0[i, pl.ds(k * 16, 16)] = jnp.zeros((16,), jnp.float32)
        return 0
    lax.fori_loop(0, MP, z_body, 0)
    base = s * RPS
    for q in range(RPS // MP):
        pltpu.sync_copy(b0, acc_sh.at[pl.ds(base + q * MP, MP)])
    plsc.subcore_barrier()

    pltpu.sync_copy(srcc_hbm.at[c, s], src_v)
    pltpu.sync_copy(dstlc_hbm.at[c, s], dlx_v)
    pltpu.sync_copy(cnt_hbm.at[c, s], cnt_v)
    cnt = jnp.max(cnt_v[pl.ds(0, 16)])

    for j in range(6):
        @pl.when(j * MP < cnt)
        def _(j=j):
            pltpu.async_copy(hs_hbm.at[src_v.at[j]], bufs[j], sems[j])

    def group(g, _):
        for j in range(6):
            b = g * 6 + j

            @pl.when(b * MP < cnt)
            def _(b=b, j=j):
                pltpu.make_async_copy(hs_hbm.at[src_v.at[b]], bufs[j],
                                      sems[j]).wait()
                pltpu.sync_copy(bufs[j], acc_sh.at[dlx_v.at[b]], add=True)
                nxt = b + 6

                @pl.when(nxt * MP < cnt)
                def _():
                    pltpu.async_copy(hs_hbm.at[src_v.at[nxt]], bufs[j],
                                     sems[j])
        return 0

    lax.fori_loop(0, 22, group, 0)
    plsc.subcore_barrier()
    pltpu.sync_copy(acc_sh.at[pl.ds(base, RPS)],
                    acc_hbm.at[c, pl.ds(base, RPS)])


_propagate = pl.kernel(
    _propagate_body,
    out_type=jax.ShapeDtypeStruct((NCORE, SLAB, H), jnp.float32),
    mesh=_MESH,
    compiler_params=pltpu.CompilerParams(
        needs_layout_passes=False, use_tc_tiling_on_sc=False),
    scratch_types=[
        pltpu.VMEM((NBP, MP), jnp.int32),
        pltpu.VMEM((NBP, MP), jnp.int32),
        pltpu.VMEM((16,), jnp.int32),
        pltpu.VMEM((MP, H), jnp.float32),
        pltpu.VMEM((MP, H), jnp.float32),
        pltpu.VMEM((MP, H), jnp.float32),
        pltpu.VMEM((MP, H), jnp.float32),
        pltpu.VMEM((MP, H), jnp.float32),
        pltpu.VMEM((MP, H), jnp.float32),
        pltpu.VMEM_SHARED((SLAB, H), jnp.float32),
        pltpu.SemaphoreType.DMA,
        pltpu.SemaphoreType.DMA,
        pltpu.SemaphoreType.DMA,
        pltpu.SemaphoreType.DMA,
        pltpu.SemaphoreType.DMA,
        pltpu.SemaphoreType.DMA,
    ],
)


def _mm_body(x_ref, w_ref, xw_ref):
    xw_ref[...] = jnp.dot(x_ref[...], w_ref[...],
                          preferred_element_type=jnp.float32)


def _mm(x, w):
    r = x.shape[0] // NPG
    return pl.pallas_call(
        _mm_body,
        grid=(r,),
        in_specs=[
            pl.BlockSpec((NPG, x.shape[1]), lambda g: (g, 0)),
            pl.BlockSpec(w.shape, lambda g: (0, 0)),
        ],
        out_specs=pl.BlockSpec((NPG, H), lambda g: (g, 0)),
        out_shape=jax.ShapeDtypeStruct((x.shape[0], H), jnp.float32),
    )(x, w)


def _scale_body(xw_ref, degp_ref, xws_ref, deg_ref):
    dp = jnp.sum(degp_ref[...], axis=0)
    deg0 = jnp.sum(dp, axis=1, keepdims=True)
    di = lax.rsqrt(deg0 + 1.0)
    xws_ref[...] = di * xw_ref[...]
    deg_ref[...] = deg0


def _scale(xw, degp):
    r = xw.shape[0] // NPG
    return pl.pallas_call(
        _scale_body,
        grid=(r,),
        in_specs=[
            pl.BlockSpec((NPG, H), lambda g: (g, 0)),
            pl.BlockSpec((NCORE, NPG, 16), lambda g: (0, g, 0)),
        ],
        out_specs=[
            pl.BlockSpec((NPG, H), lambda g: (g, 0)),
            pl.BlockSpec((NPG, 1), lambda g: (g, 0)),
        ],
        out_shape=[
            jax.ShapeDtypeStruct((xw.shape[0], H), jnp.float32),
            jax.ShapeDtypeStruct((xw.shape[0], 1), jnp.float32),
        ],
    )(xw, degp)


def _gcn_epi_body(acc_ref, xws_ref, deg_ref, b_ref, h_ref, hs_ref):
    deg0 = deg_ref[...]
    di1 = lax.rsqrt(deg0 + 1.0)
    di0 = jnp.where(deg0 > 0.0, lax.rsqrt(jnp.maximum(deg0, 1e-30)), 0.0)
    h = jnp.maximum(di1 * (acc_ref[0] + xws_ref[...]) + b_ref[...], 0.0)
    h_ref[...] = h
    hs_ref[...] = di0 * h


def _gcn_epilogue(acc, xws, deg0, b):
    return pl.pallas_call(
        _gcn_epi_body,
        grid=(B,),
        in_specs=[
            pl.BlockSpec((1, NPG, H), lambda g: (g // 5, g % 5, 0)),
            pl.BlockSpec((NPG, H), lambda g: (g, 0)),
            pl.BlockSpec((NPG, 1), lambda g: (g, 0)),
            pl.BlockSpec((1, H), lambda g: (0, 0)),
        ],
        out_specs=[
            pl.BlockSpec((NPG, H), lambda g: (g, 0)),
            pl.BlockSpec((NPG, H), lambda g: (g, 0)),
        ],
        out_shape=[
            jax.ShapeDtypeStruct((N, H), jnp.float32),
            jax.ShapeDtypeStruct((N, H), jnp.float32),
        ],
    )(acc, xws, deg0, b)


def _score_pool_body(k, h_ref, acc_ref, deg_ref, mcol_ref, mrow_ref,
                     arow_ref, acol_ref, gmp_ref, gap_ref):
    deg0 = deg_ref[...]
    di0 = jnp.where(deg0 > 0.0, lax.rsqrt(jnp.maximum(deg0, 1e-30)), 0.0)
    h = h_ref[...]
    prop = jnp.abs(h - di0 * acc_ref[0])
    ones_row = jnp.ones((1, H), jnp.float32)
    s_col = jnp.sum(prop, axis=1, keepdims=True)
    s_row = lax.dot_general(ones_row, prop, (((1,), (1,)), ((), ())),
                            preferred_element_type=jnp.float32)
    ninf = jnp.float32(-jnp.inf)
    s_col = jnp.where(mcol_ref[...] > 0.0, s_col, ninf)
    s_row = jnp.where(mrow_ref[0] > 0.0, s_row, ninf)
    i0 = lax.broadcasted_iota(jnp.int32, (NPG, NPG), 0)
    i1 = lax.broadcasted_iota(jnp.int32, (NPG, NPG), 1)
    beats = (s_row > s_col) | ((s_row == s_col) & (i1 < i0))
    pf = jnp.where(beats, 1.0, 0.0)
    ones_col = jnp.ones((NPG, 1), jnp.float32)
    rank_col = jnp.dot(pf, ones_col, preferred_element_type=jnp.float32)
    a_col = jnp.where(rank_col < float(k), 1.0, 0.0)
    ones_1n = jnp.ones((1, NPG), jnp.float32)
    colsum = lax.dot_general(ones_1n, pf, (((1,), (0,)), ((), ())),
                             preferred_element_type=jnp.float32)
    rank_row = float(NPG - 1) - colsum
    a_row = jnp.where(rank_row < float(k), 1.0, 0.0)
    arow_ref[0] = a_row
    acol_ref[...] = a_col
    gmp_ref[0] = jnp.max(jnp.where(a_col > 0.0, h, ninf), axis=0,
                         keepdims=True)
    gap_ref[0] = jnp.sum(jnp.where(a_col > 0.0, h, 0.0), axis=0,
                         keepdims=True) * (1.0 / k)


def _score_pool(k, h, acc, deg0, mcol, mrow):
    return pl.pallas_call(
        functools.partial(_score_pool_body, k),
        grid=(B,),
        in_specs=[
            pl.BlockSpec((NPG, H), lambda g: (g, 0)),
            pl.BlockSpec((1, NPG, H), lambda g: (g // 5, g % 5, 0)),
            pl.BlockSpec((NPG, 1), lambda g: (g, 0)),
            pl.BlockSpec((NPG, 1), lambda g: (g, 0)),
            pl.BlockSpec((1, 1, NPG), lambda g: (g, 0, 0)),
        ],
        out_specs=[
            pl.BlockSpec((1, 1, NPG), lambda g: (g, 0, 0)),
            pl.BlockSpec((NPG, 1), lambda g: (g, 0)),
            pl.BlockSpec((1, 1, H), lambda g: (g, 0, 0)),
            pl.BlockSpec((1, 1, H), lambda g: (g, 0, 0)),
        ],
        out_shape=[
            jax.ShapeDtypeStruct((B, 1, NPG), jnp.float32),
            jax.ShapeDtypeStruct((N, 1), jnp.float32),
            jax.ShapeDtypeStruct((B, 1, H), jnp.float32),
            jax.ShapeDtypeStruct((B, 1, H), jnp.float32),
        ],
    )(h, acc, deg0, mcol, mrow)


def _final_gcn_body(acc_ref, xws_ref, deg_ref, b_ref, acol_ref,
                    gmp_ref, gap_ref):
    di1 = lax.rsqrt(deg_ref[...] + 1.0)
    h = jnp.maximum(di1 * (acc_ref[0] + xws_ref[...]) + b_ref[...], 0.0)
    a_col = acol_ref[...]
    ninf = jnp.float32(-jnp.inf)
    gmp_ref[0] = jnp.max(jnp.where(a_col > 0.0, h, ninf), axis=0,
                         keepdims=True)
    gap_ref[0] = jnp.sum(jnp.where(a_col > 0.0, h, 0.0), axis=0,
                         keepdims=True) * (1.0 / K2)


def _final_gcn(acc, xws, deg0, b, acol):
    return pl.pallas_call(
        _final_gcn_body,
        grid=(B,),
        in_specs=[
            pl.BlockSpec((1, NPG, H), lambda g: (g // 5, g % 5, 0)),
            pl.BlockSpec((NPG, H), lambda g: (g, 0)),
            pl.BlockSpec((NPG, 1), lambda g: (g, 0)),
            pl.BlockSpec((1, H), lambda g: (0, 0)),
            pl.BlockSpec((NPG, 1), lambda g: (g, 0)),
        ],
        out_specs=[
            pl.BlockSpec((1, 1, H), lambda g: (g, 0, 0)),
            pl.BlockSpec((1, 1, H), lambda g: (g, 0, 0)),
        ],
        out_shape=[
            jax.ShapeDtypeStruct((B, 1, H), jnp.float32),
            jax.ShapeDtypeStruct((B, 1, H), jnp.float32),
        ],
    )(acc, xws, deg0, b, acol)


def _head_body(g1, a1, g2, a2, g3, a3, skew_ref, wsk_ref, bsk_ref,
               wl1_ref, bl1_ref, wl2_ref, bl2_ref, wl3_ref, bl3_ref, out_ref):
    r = jnp.maximum
    zmx = r(g1[...], 0.0) + r(g2[...], 0.0) + r(g3[...], 0.0)
    zmn = r(a1[...], 0.0) + r(a2[...], 0.0) + r(a3[...], 0.0)
    xskew = r(jnp.dot(skew_ref[...], wsk_ref[...],
                      preferred_element_type=jnp.float32) + bsk_ref[...], 0.0)
    z = jnp.concatenate([zmx, zmn, xskew], axis=1)
    z = r(jnp.dot(z, wl1_ref[...], preferred_element_type=jnp.float32)
          + bl1_ref[...], 0.0)
    z = r(jnp.dot(z, wl2_ref[...], preferred_element_type=jnp.float32)
          + bl2_ref[...], 0.0)
    o = jnp.dot(z, wl3_ref[...], preferred_element_type=jnp.float32) \
        + bl3_ref[...]
    m = jnp.max(o, axis=1, keepdims=True)
    lse = jnp.log(jnp.sum(jnp.exp(o - m), axis=1, keepdims=True)) + m
    out_ref[...] = o - lse


def _head(g1, a1, g2, a2, g3, a3, skew, wsk, bsk, wl1, bl1, wl2, bl2,
          wl3, bl3):
    args = (g1, a1, g2, a2, g3, a3, skew, wsk, bsk, wl1, bl1, wl2, bl2,
            wl3, bl3)
    return pl.pallas_call(
        _head_body,
        in_specs=[pl.BlockSpec(a.shape, lambda: (0, 0)) for a in args],
        out_specs=pl.BlockSpec((B, C), lambda: (0, 0)),
        out_shape=jax.ShapeDtypeStruct((B, C), jnp.float32),
    )(*args)


def kernel(x, edge_index, batch, skew, W1, b1, Wsk, bsk, W2, b2, W3, b3,
           Wl1, bl1, Wl2, bl2, Wl3, bl3):
    del batch
    src = edge_index[0]
    dst = edge_index[1]
    src_r = src.reshape(NCORE, NSUB, NB, M)
    dst_r = dst.reshape(NCORE, NSUB, NB, M)
    ones_n = jnp.ones((N,), jnp.float32)
    ones_col = jnp.ones((N, 1), jnp.float32)
    ones_row = jnp.ones((B, 1, NPG), jnp.float32)
    b1r = b1.reshape(1, H)
    b2r = b2.reshape(1, H)
    b3r = b3.reshape(1, H)
    r4 = lambda t: t.reshape(NCORE, NSUB, NBP, MP)

    xw1 = _mm(x, W1)
    degp1, srcc1, dstlc1, cnt1 = _edge_pass(ones_n, src_r, dst_r)
    srcc1, dstlc1 = r4(srcc1), r4(dstlc1)
    xw1s, deg01 = _scale(xw1, degp1)
    acc1 = _propagate(xw1s, srcc1, dstlc1, cnt1)
    h1, h1s = _gcn_epilogue(acc1, xw1s, deg01, b1r)
    acc2 = _propagate(h1s, srcc1, dstlc1, cnt1)
    a2row, a2col, gmp1, gap1 = _score_pool(K1, h1, acc2, deg01,
                                           ones_col, ones_row)

    xw2 = _mm(h1, W2)
    degp2, srcc2, dstlc2, cnt2 = _edge_pass(a2col.reshape(N), src_r, dst_r)
    srcc2, dstlc2 = r4(srcc2), r4(dstlc2)
    xw2s, deg02 = _scale(xw2, degp2)
    acc3 = _propagate(xw2s, srcc2, dstlc2, cnt2)
    h2, h2s = _gcn_epilogue(acc3, xw2s, deg02, b2r)
    acc4 = _propagate(h2s, srcc2, dstlc2, cnt2)
    a3row, a3col, gmp2, gap2 = _score_pool(K2, h2, acc4, deg02, a2col, a2row)

    xw3 = _mm(h2, W3)
    degp3, srcc3, dstlc3, cnt3 = _edge_pass(a3col.reshape(N), src_r, dst_r)
    srcc3, dstlc3 = r4(srcc3), r4(dstlc3)
    xw3s, deg03 = _scale(xw3, degp3)
    acc5 = _propagate(xw3s, srcc3, dstlc3, cnt3)
    gmp3, gap3 = _final_gcn(acc5, xw3s, deg03, b3r, a3col)

    rs = lambda t: t.reshape(B, H)
    return _head(rs(gmp1), rs(gap1), rs(gmp2), rs(gap2), rs(gmp3), rs(gap3), skew,
                 Wsk, bsk.reshape(1, H), Wl1, bl1.reshape(1, H),
                 Wl2, bl2.reshape(1, SK), Wl3, bl3.reshape(1, C))
